# Initial kernel scaffold; baseline (speedup 1.0000x reference)
import jax
import jax.numpy as jnp
from jax.experimental import pallas as pl


def kernel(x, table):
    raise NotImplementedError("write your pallas kernel here")


# speedup vs baseline: 1.0943x; 1.0943x over previous
import functools

import jax
import jax.numpy as jnp
from jax import lax
from jax.experimental import pallas as pl
from jax.experimental.pallas import tpu as pltpu
from jax.experimental.pallas import tpu_sc as plsc

D = 32
SUB = 128
K = 8
CHUNK = K * SUB
NC = 2
NS = 16
NW = NC * NS


def _sc_gather(table, idx2d):
    n_rows = idx2d.shape[0]
    rows_per_w = n_rows // NW
    n_chunks = rows_per_w // K
    b_flat = n_rows * SUB

    mesh = plsc.VectorSubcoreMesh(core_axis_name="c", subcore_axis_name="s")

    @functools.partial(
        pl.kernel,
        mesh=mesh,
        out_type=jax.ShapeDtypeStruct((b_flat, D), jnp.float32),
        scratch_types=[
            pltpu.VMEM((K, SUB), jnp.int32),
            pltpu.VMEM((CHUNK, D), jnp.float32),
            pltpu.SemaphoreType.DMA,
        ],
        compiler_params=pltpu.CompilerParams(use_tc_tiling_on_sc=False),
    )
    def k(table_hbm, idx_hbm, out_hbm, idx_v, rows_v, gsem):
        wid = lax.axis_index("s") * NC + lax.axis_index("c")
        row_base = wid * rows_per_w

        def body(i, carry):
            coff = row_base + i * K
            pltpu.sync_copy(idx_hbm.at[pl.ds(coff, K), :], idx_v)
            for j in range(K):
                pltpu.async_copy(
                    table_hbm.at[idx_v.at[j]],
                    rows_v.at[pl.ds(j * SUB, SUB), :],
                    gsem,
                )
            for j in range(K):
                pltpu.make_async_copy(
                    table_hbm.at[idx_v.at[j]],
                    rows_v.at[pl.ds(j * SUB, SUB), :],
                    gsem,
                ).wait()
            pltpu.sync_copy(rows_v, out_hbm.at[pl.ds(coff * SUB, CHUNK), :])
            return carry

        lax.fori_loop(0, n_chunks, body, 0)

    return k(table, idx2d)


def kernel(x, table):
    idx = x.reshape(-1).astype(jnp.int32)
    idx2d = idx.reshape(-1, SUB)
    out = _sc_gather(table, idx2d)
    return out.reshape(x.shape + (table.shape[1],))


# speedup vs baseline: 1.1099x; 1.0142x over previous
import functools

import jax
import jax.numpy as jnp
from jax import lax
from jax.experimental import pallas as pl
from jax.experimental.pallas import tpu as pltpu
from jax.experimental.pallas import tpu_sc as plsc

D = 32
SUB = 128
K = 8
CHUNK = K * SUB
NC = 2
NS = 16
NW = NC * NS


def _sc_gather(table, idx2d):
    n_rows = idx2d.shape[0]
    rows_per_w = n_rows // NW
    n_chunks = rows_per_w // K
    n_pairs = (n_chunks - 1) // 2
    b_flat = n_rows * SUB

    mesh = plsc.VectorSubcoreMesh(core_axis_name="c", subcore_axis_name="s")

    @functools.partial(
        pl.kernel,
        mesh=mesh,
        out_type=jax.ShapeDtypeStruct((b_flat, D), jnp.float32),
        scratch_types=[
            pltpu.VMEM((2, K, SUB), jnp.int32),
            pltpu.VMEM((CHUNK, D), jnp.float32),
            pltpu.VMEM((CHUNK, D), jnp.float32),
            pltpu.SemaphoreType.DMA,
            pltpu.SemaphoreType.DMA,
            pltpu.SemaphoreType.DMA,
            pltpu.SemaphoreType.DMA,
            pltpu.SemaphoreType.DMA,
            pltpu.SemaphoreType.DMA,
        ],
        compiler_params=pltpu.CompilerParams(use_tc_tiling_on_sc=False),
    )
    def k(table_hbm, idx_hbm, out_hbm, idx_v, rows0, rows1,
          isem0, isem1, gsem0, gsem1, wsem0, wsem1):
        wid = lax.axis_index("s") * NC + lax.axis_index("c")
        row_base = wid * rows_per_w
        rows_v = (rows0, rows1)
        isems = (isem0, isem1)
        gsems = (gsem0, gsem1)
        wsems = (wsem0, wsem1)

        def idx_src(i):
            return idx_hbm.at[pl.ds(row_base + i * K, K), :]

        def out_dst(i):
            return out_hbm.at[pl.ds((row_base + i * K) * SUB, CHUNK), :]

        def fire_gathers(b):
            for j in range(K):
                pltpu.async_copy(
                    table_hbm.at[idx_v.at[b, j]],
                    rows_v[b].at[pl.ds(j * SUB, SUB), :],
                    gsems[b],
                )

        def drain_gathers(b):
            for j in range(K):
                pltpu.make_async_copy(
                    table_hbm.at[idx_v.at[b, j]],
                    rows_v[b].at[pl.ds(j * SUB, SUB), :],
                    gsems[b],
                ).wait()

        pltpu.async_copy(idx_src(0), idx_v.at[0], isem0)
        pltpu.async_copy(idx_src(1), idx_v.at[1], isem1)

        pltpu.make_async_copy(idx_src(0), idx_v.at[0], isem0).wait()
        fire_gathers(0)
        drain_gathers(0)
        pltpu.async_copy(idx_src(2), idx_v.at[0], isem0)
        pltpu.async_copy(rows_v[0], out_dst(0), wsems[0])

        def pair_body(p, carry):
            for b, off in ((1, 1), (0, 2)):
                i = p * 2 + off
                pltpu.make_async_copy(idx_src(i), idx_v.at[b], isems[b]).wait()
                if b == 0:
                    pltpu.make_async_copy(
                        rows_v[b], out_dst(i - 2), wsems[b]).wait()
                else:
                    @pl.when(p >= 1)
                    def _():
                        pltpu.make_async_copy(
                            rows_v[b], out_dst(i - 2), wsems[b]).wait()
                fire_gathers(b)
                drain_gathers(b)
                @pl.when(p < n_pairs - 1)
                def _():
                    pltpu.async_copy(idx_src(i + 2), idx_v.at[b], isems[b])
                pltpu.async_copy(rows_v[b], out_dst(i), wsems[b])
            return carry

        lax.fori_loop(0, n_pairs, pair_body, 0)

        pltpu.make_async_copy(rows_v[1], out_dst(n_chunks - 2), wsems[1]).wait()
        pltpu.make_async_copy(rows_v[0], out_dst(n_chunks - 1), wsems[0]).wait()

    return k(table, idx2d)


def kernel(x, table):
    idx = x.reshape(-1).astype(jnp.int32)
    idx2d = idx.reshape(-1, SUB)
    out = _sc_gather(table, idx2d)
    return out.reshape(x.shape + (table.shape[1],))


# speedup vs baseline: 1.9308x; 1.7396x over previous
import functools

import jax
import jax.numpy as jnp
from jax import lax
from jax.experimental import pallas as pl
from jax.experimental.pallas import tpu as pltpu
from jax.experimental.pallas import tpu_sc as plsc

D = 32
SUB = 128
K = 8
CHUNK = K * SUB
NC = 2
NS = 16
NW = NC * NS


def _sc_gather(table, idx2d):
    n_rows = idx2d.shape[0]
    rows_per_w = n_rows // NW
    n_chunks = rows_per_w // K
    n_pairs = (n_chunks - 1) // 2
    b_flat = n_rows * SUB

    mesh = plsc.VectorSubcoreMesh(core_axis_name="c", subcore_axis_name="s")

    @functools.partial(
        pl.kernel,
        mesh=mesh,
        out_type=jax.ShapeDtypeStruct((b_flat, D), jnp.float32),
        scratch_types=[
            pltpu.VMEM((2, K, SUB), jnp.int32),
            pltpu.VMEM((CHUNK, D), jnp.float32),
            pltpu.VMEM((CHUNK, D), jnp.float32),
            pltpu.SemaphoreType.DMA,
            pltpu.SemaphoreType.DMA,
            pltpu.SemaphoreType.DMA,
            pltpu.SemaphoreType.DMA,
            pltpu.SemaphoreType.DMA,
            pltpu.SemaphoreType.DMA,
        ],
        compiler_params=pltpu.CompilerParams(use_tc_tiling_on_sc=False),
    )
    def k(table_hbm, idx_hbm, out_hbm, idx_v, rows0, rows1,
          isem0, isem1, gsem0, gsem1, wsem0, wsem1):
        wid = lax.axis_index("s") * NC + lax.axis_index("c")
        row_base = wid * rows_per_w
        rows_v = (rows0, rows1)
        isems = (isem0, isem1)
        gsems = (gsem0, gsem1)
        wsems = (wsem0, wsem1)

        def idx_src(i):
            return idx_hbm.at[pl.ds(row_base + i * K, K), :]

        def out_dst(i):
            return out_hbm.at[pl.ds((row_base + i * K) * SUB, CHUNK), :]

        def fire_gathers(b):
            for j in range(K):
                pltpu.async_copy(
                    table_hbm.at[idx_v.at[b, j]],
                    rows_v[b].at[pl.ds(j * SUB, SUB), :],
                    gsems[b],
                )

        def drain_gathers(b):
            for j in range(K):
                pltpu.make_async_copy(
                    table_hbm.at[idx_v.at[b, j]],
                    rows_v[b].at[pl.ds(j * SUB, SUB), :],
                    gsems[b],
                ).wait()

        pltpu.async_copy(idx_src(0), idx_v.at[0], isem0)
        pltpu.async_copy(idx_src(1), idx_v.at[1], isem1)

        pltpu.make_async_copy(idx_src(0), idx_v.at[0], isem0).wait()
        fire_gathers(0)
        drain_gathers(0)
        pltpu.async_copy(idx_src(2), idx_v.at[0], isem0)
        pltpu.async_copy(rows_v[0], out_dst(0), wsems[0])

        def pair_body(p, carry):
            for b, off in ((1, 1), (0, 2)):
                i = p * 2 + off
                pltpu.make_async_copy(idx_src(i), idx_v.at[b], isems[b]).wait()
                if b == 0:
                    pltpu.make_async_copy(
                        rows_v[b], out_dst(i - 2), wsems[b]).wait()
                else:
                    @pl.when(p >= 1)
                    def _():
                        pltpu.make_async_copy(
                            rows_v[b], out_dst(i - 2), wsems[b]).wait()
                fire_gathers(b)
                drain_gathers(b)
                @pl.when(p < n_pairs - 1)
                def _():
                    pltpu.async_copy(idx_src(i + 2), idx_v.at[b], isems[b])
                pltpu.async_copy(rows_v[b], out_dst(i), wsems[b])
            return carry

        lax.fori_loop(0, n_pairs, pair_body, 0)

        pltpu.make_async_copy(rows_v[1], out_dst(n_chunks - 2), wsems[1]).wait()
        pltpu.make_async_copy(rows_v[0], out_dst(n_chunks - 1), wsems[0]).wait()

    return k(table, idx2d)


def kernel(x, table):
    b, s = x.shape
    idx2d = x.T.astype(jnp.int32).reshape(-1, SUB)
    out = _sc_gather(table, idx2d)
    return jnp.swapaxes(out.reshape(s, b, table.shape[1]), 0, 1)
